# baseline (speedup 1.0000x reference)
import functools

import jax
import jax.numpy as jnp
from jax import lax
from jax.experimental import pallas as pl
from jax.experimental.pallas import tpu as pltpu
from jax.experimental.pallas import tpu_sc as plsc

KC = 1024
DD = 256
NTOK = 32 * 1024

TOK_BLK = 512
NB = NTOK // TOK_BLK

NC, NS = 2, 16
NW = NC * NS
CH = 128
NCH = NTOK // NW // CH


def _argmin_body(xsq_ref, wsq_ref, x_ref, w_ref, idx_ref):
    x = x_ref[...]
    w = w_ref[...]
    mm = lax.dot_general(x, w, (((1,), (1,)), ((), ())),
                         preferred_element_type=jnp.float32)
    d = (xsq_ref[...] + wsq_ref[...]) - 2.0 * mm
    m = jnp.min(d, axis=1, keepdims=True)
    ks = lax.broadcasted_iota(jnp.int32, d.shape, 1)
    idx = jnp.min(jnp.where(d == m, ks, jnp.int32(KC)), axis=1)
    idx_ref[...] = idx.reshape(1, 1, TOK_BLK)


_argmin_call = pl.pallas_call(
    _argmin_body,
    grid=(NB,),
    in_specs=[
        pl.BlockSpec((TOK_BLK, 1), lambda i: (i, 0)),
        pl.BlockSpec((1, KC), lambda i: (0, 0)),
        pl.BlockSpec((TOK_BLK, DD), lambda i: (i, 0)),
        pl.BlockSpec((KC, DD), lambda i: (0, 0)),
    ],
    out_specs=pl.BlockSpec((1, 1, TOK_BLK), lambda i: (i, 0, 0)),
    out_shape=jax.ShapeDtypeStruct((NB, 1, TOK_BLK), jnp.int32),
)


@functools.lru_cache(maxsize=None)
def _make_gather():
    @functools.partial(
        pl.kernel,
        out_type=jax.ShapeDtypeStruct((NTOK, DD), jnp.float32),
        mesh=plsc.VectorSubcoreMesh(core_axis_name="c", subcore_axis_name="s"),
        scratch_types=[
            pltpu.VMEM((NCH, CH), jnp.int32),
            pltpu.VMEM((CH, DD), jnp.float32),
            pltpu.VMEM((CH, DD), jnp.float32),
            pltpu.SemaphoreType.DMA,
            pltpu.SemaphoreType.DMA,
        ],
    )
    def _gather_call(idx_hbm, w_hbm, out_hbm, idx_v, buf_a, buf_b, sem_a, sem_b):
        wid = lax.axis_index("s") * NC + lax.axis_index("c")
        base = wid * (NCH * CH)
        pltpu.sync_copy(idx_hbm.at[wid], idx_v)
        bufs = (buf_a, buf_b)
        sems = (sem_a, sem_b)
        copies = [None, None]
        copies[0] = pltpu.async_copy(w_hbm.at[idx_v.at[0]], bufs[0], sems[0])
        for c in range(NCH):
            p = c % 2
            if c + 1 < NCH:
                q = (c + 1) % 2
                copies[q] = pltpu.async_copy(w_hbm.at[idx_v.at[c + 1]], bufs[q], sems[q])
            copies[p].wait()
            pltpu.sync_copy(bufs[p], out_hbm.at[pl.ds(base + c * CH, CH)])

    return _gather_call


def kernel(latents, W):
    shape = latents.shape
    flat = latents.reshape(-1, W.shape[1])
    xsq = jnp.sum(flat ** 2, axis=1, keepdims=True)
    wsq = jnp.sum(W ** 2, axis=1)[None, :]
    idx = _argmin_call(xsq, wsq, flat, W)
    q = _make_gather()(idx.reshape(NW, NCH, CH), W)
    return q.reshape(shape)


# speedup vs baseline: 1.0367x; 1.0367x over previous
import functools

import jax
import jax.numpy as jnp
from jax import lax
from jax.experimental import pallas as pl
from jax.experimental.pallas import tpu as pltpu
from jax.experimental.pallas import tpu_sc as plsc

KC = 1024
DD = 256
NTOK = 32 * 1024

TOK_BLK = 512
NB = NTOK // TOK_BLK

NC, NS = 2, 16
NW = NC * NS
CH = 128
NCH = NTOK // NW // CH


def _argmin_body(xsq_ref, wsq_ref, rk_ref, x_ref, w_ref, idx_ref):
    x = x_ref[...]
    w = w_ref[...]
    mm2 = lax.dot_general(x + x, w, (((1,), (1,)), ((), ())),
                          preferred_element_type=jnp.float32)
    d = (xsq_ref[...] + wsq_ref[...]) - mm2
    m = jnp.min(d, axis=1, keepdims=True)
    r = jnp.max(jnp.where(d == m, rk_ref[...], -1.0), axis=1, keepdims=True)
    idx_ref[...] = (jnp.float32(KC - 1) - r).astype(jnp.int32)


_argmin_call = pl.pallas_call(
    _argmin_body,
    grid=(NB,),
    in_specs=[
        pl.BlockSpec((TOK_BLK, 1), lambda i: (i, 0)),
        pl.BlockSpec((1, KC), lambda i: (0, 0)),
        pl.BlockSpec((1, KC), lambda i: (0, 0)),
        pl.BlockSpec((TOK_BLK, DD), lambda i: (i, 0)),
        pl.BlockSpec((KC, DD), lambda i: (0, 0)),
    ],
    out_specs=pl.BlockSpec((TOK_BLK, 1), lambda i: (i, 0)),
    out_shape=jax.ShapeDtypeStruct((NTOK, 1), jnp.int32),
)


@functools.lru_cache(maxsize=None)
def _make_gather():
    @functools.partial(
        pl.kernel,
        out_type=jax.ShapeDtypeStruct((NTOK, DD), jnp.float32),
        mesh=plsc.VectorSubcoreMesh(core_axis_name="c", subcore_axis_name="s"),
        scratch_types=[
            pltpu.VMEM((NCH, CH), jnp.int32),
            pltpu.VMEM((CH, DD), jnp.float32),
            pltpu.VMEM((CH, DD), jnp.float32),
            pltpu.SemaphoreType.DMA,
            pltpu.SemaphoreType.DMA,
        ],
    )
    def _gather_call(idx_hbm, w_hbm, out_hbm, idx_v, buf_a, buf_b, sem_a, sem_b):
        wid = lax.axis_index("s") * NC + lax.axis_index("c")
        base = wid * (NCH * CH)
        pltpu.sync_copy(idx_hbm.at[wid], idx_v)
        bufs = (buf_a, buf_b)
        sems = (sem_a, sem_b)
        copies = [None, None]
        copies[0] = pltpu.async_copy(w_hbm.at[idx_v.at[0]], bufs[0], sems[0])
        for c in range(NCH):
            p = c % 2
            if c + 1 < NCH:
                q = (c + 1) % 2
                copies[q] = pltpu.async_copy(w_hbm.at[idx_v.at[c + 1]], bufs[q], sems[q])
            copies[p].wait()
            pltpu.sync_copy(bufs[p], out_hbm.at[pl.ds(base + c * CH, CH)])

    return _gather_call


def kernel(latents, W):
    shape = latents.shape
    flat = latents.reshape(-1, W.shape[1])
    xsq = jnp.sum(flat ** 2, axis=1, keepdims=True)
    wsq = jnp.sum(W ** 2, axis=1)[None, :]
    rk = jnp.float32(KC - 1) - lax.iota(jnp.float32, KC)[None, :]
    idx = _argmin_call(xsq, wsq, rk, flat, W)
    q = _make_gather()(idx.reshape(NW, NCH, CH), W)
    return q.reshape(shape)


# speedup vs baseline: 1.2027x; 1.1601x over previous
import functools

import jax
import jax.numpy as jnp
from jax import lax
from jax.experimental import pallas as pl
from jax.experimental.pallas import tpu as pltpu
from jax.experimental.pallas import tpu_sc as plsc

KC = 1024
DD = 256
NTOK = 32 * 1024

TOK_BLK = 512
NB = NTOK // TOK_BLK

NC, NS = 2, 16
NW = NC * NS
CH = 128
NCH = NTOK // NW // CH


def _argmin_body(xsq_ref, wsq_ref, rk_ref, x_ref, w_ref, idx_ref):
    x = x_ref[...]
    w = w_ref[...]
    mm2 = lax.dot_general(w + w, x, (((1,), (1,)), ((), ())),
                          preferred_element_type=jnp.float32)
    d = (xsq_ref[...].reshape(1, TOK_BLK) + wsq_ref[...]) - mm2
    m = jnp.min(d, axis=0, keepdims=True)
    r = jnp.max(jnp.where(d == m, rk_ref[...], -1.0), axis=0, keepdims=True)
    idx_ref[...] = (jnp.float32(KC - 1) - r).astype(jnp.int32).reshape(1, 1, TOK_BLK)


_argmin_call = pl.pallas_call(
    _argmin_body,
    grid=(NB,),
    in_specs=[
        pl.BlockSpec((1, 1, TOK_BLK), lambda i: (i, 0, 0)),
        pl.BlockSpec((KC, 1), lambda i: (0, 0)),
        pl.BlockSpec((KC, 1), lambda i: (0, 0)),
        pl.BlockSpec((TOK_BLK, DD), lambda i: (i, 0)),
        pl.BlockSpec((KC, DD), lambda i: (0, 0)),
    ],
    out_specs=pl.BlockSpec((1, 1, TOK_BLK), lambda i: (i, 0, 0)),
    out_shape=jax.ShapeDtypeStruct((NB, 1, TOK_BLK), jnp.int32),
)


@functools.lru_cache(maxsize=None)
def _make_gather():
    @functools.partial(
        pl.kernel,
        out_type=jax.ShapeDtypeStruct((NTOK, DD), jnp.float32),
        mesh=plsc.VectorSubcoreMesh(core_axis_name="c", subcore_axis_name="s"),
        scratch_types=[
            pltpu.VMEM((NCH, CH), jnp.int32),
            pltpu.VMEM((CH, DD), jnp.float32),
            pltpu.VMEM((CH, DD), jnp.float32),
            pltpu.SemaphoreType.DMA,
            pltpu.SemaphoreType.DMA,
        ],
    )
    def _gather_call(idx_hbm, w_hbm, out_hbm, idx_v, buf_a, buf_b, sem_a, sem_b):
        wid = lax.axis_index("s") * NC + lax.axis_index("c")
        base = wid * (NCH * CH)
        pltpu.sync_copy(idx_hbm.at[wid], idx_v)
        bufs = (buf_a, buf_b)
        sems = (sem_a, sem_b)
        copies = [None, None]
        copies[0] = pltpu.async_copy(w_hbm.at[idx_v.at[0]], bufs[0], sems[0])
        for c in range(NCH):
            p = c % 2
            if c + 1 < NCH:
                q = (c + 1) % 2
                copies[q] = pltpu.async_copy(w_hbm.at[idx_v.at[c + 1]], bufs[q], sems[q])
            copies[p].wait()
            pltpu.sync_copy(bufs[p], out_hbm.at[pl.ds(base + c * CH, CH)])

    return _gather_call


def kernel(latents, W):
    shape = latents.shape
    flat = latents.reshape(-1, W.shape[1])
    xsq = jnp.sum(flat ** 2, axis=1).reshape(NB, 1, TOK_BLK)
    wsq = jnp.sum(W ** 2, axis=1)[:, None]
    rk = (jnp.float32(KC - 1) - lax.iota(jnp.float32, KC))[:, None]
    idx = _argmin_call(xsq, wsq, rk, flat, W)
    q = _make_gather()(idx.reshape(NW, NCH, CH), W)
    return q.reshape(shape)
